# baseline (speedup 1.0000x reference)
import functools

import numpy as np
import jax
import jax.numpy as jnp
from jax import lax
from jax.experimental import pallas as pl
from jax.experimental.pallas import tpu as pltpu
from jax.experimental.pallas import tpu_sc as plsc

_B, _C, _D = 4096, 14, 1024
_ROWS = _B * _C
_NW = 32
_STRIPE = _B // _NW

_PERM = (4, 2, 5, 3, 0, 1)
_SRC = list(range(_C))
_SRC[0:6] = _PERM
_SRC[7:13] = [7 + p for p in _PERM]

_mesh = plsc.VectorSubcoreMesh(core_axis_name="c", subcore_axis_name="s")


@functools.partial(
    pl.kernel,
    mesh=_mesh,
    out_type=jax.ShapeDtypeStruct((_ROWS, _D), jnp.float32),
    scratch_types=[pltpu.SemaphoreType.DMA],
)
def _shuffle(in_hbm, out_hbm, sem):
    wid = lax.axis_index("s") * 2 + lax.axis_index("c")
    base = wid * _STRIPE
    for p in range(_C):
        pltpu.async_copy(
            in_hbm.at[pl.ds(_SRC[p] * _B + base, _STRIPE)],
            out_hbm.at[pl.ds(p * _B + base, _STRIPE)],
            sem,
        )
    for p in range(_C):
        pltpu.make_async_copy(
            in_hbm.at[pl.ds(_SRC[p] * _B + base, _STRIPE)],
            out_hbm.at[pl.ds(p * _B + base, _STRIPE)],
            sem,
        ).wait()


def kernel(chromatogram_batch):
    x = chromatogram_batch.transpose(1, 0, 2).reshape(_ROWS, _D)
    out = _shuffle(x)
    return out.reshape(_C, _B, _D).transpose(1, 0, 2)


# speedup vs baseline: 38.1076x; 38.1076x over previous
import functools

import numpy as np
import jax
import jax.numpy as jnp
from jax import lax
from jax.experimental import pallas as pl
from jax.experimental.pallas import tpu as pltpu
from jax.experimental.pallas import tpu_sc as plsc

_B, _C, _D = 4096, 14, 1024
_ROWS = _B * _C
_NW = 32
_RPW = _ROWS // _NW
_CHUNK = 56
_NCH = _RPW // _CHUNK


@functools.lru_cache(maxsize=None)
def _row_index_host() -> np.ndarray:
    perm = np.array([4, 2, 5, 3, 0, 1])
    src = np.arange(_C)
    src[0:6] = perm
    src[7:13] = 7 + perm
    r = np.arange(_ROWS)
    idx = src[r // _B] * _B + (r % _B)
    return idx.astype(np.int32).reshape(_NW * _NCH, _CHUNK)


_mesh = plsc.VectorSubcoreMesh(core_axis_name="c", subcore_axis_name="s")


@functools.partial(
    pl.kernel,
    mesh=_mesh,
    out_type=jax.ShapeDtypeStruct((_ROWS, _D), jnp.float32),
    scratch_types=[
        pltpu.VMEM((_NCH, _CHUNK), jnp.int32),
        pltpu.VMEM((_CHUNK, _D), jnp.float32),
        pltpu.VMEM((_CHUNK, _D), jnp.float32),
        pltpu.SemaphoreType.DMA,
        pltpu.SemaphoreType.DMA,
        pltpu.SemaphoreType.DMA,
        pltpu.SemaphoreType.DMA,
    ],
)
def _shuffle(in_hbm, idx_hbm, out_hbm, idx_v, buf0, buf1, gsem0, gsem1,
             ssem0, ssem1):
    wid = lax.axis_index("s") * 2 + lax.axis_index("c")
    pltpu.sync_copy(idx_hbm.at[pl.ds(wid * _NCH, _NCH)], idx_v)
    base = wid * _RPW

    def gather(j, buf, sem):
        return pltpu.async_copy(in_hbm.at[idx_v.at[j]], buf, sem)

    def store(j, buf, sem):
        return pltpu.async_copy(
            buf, out_hbm.at[pl.ds(base + j * _CHUNK, _CHUNK)], sem)

    gather(0, buf0, gsem0)
    gather(1, buf1, gsem1)

    @pl.loop(0, _NCH - 2, step=2)
    def _steady(jj):
        pltpu.make_async_copy(in_hbm.at[idx_v.at[jj]], buf0, gsem0).wait()
        s0 = store(jj, buf0, ssem0)
        pltpu.make_async_copy(in_hbm.at[idx_v.at[jj + 1]], buf1, gsem1).wait()
        s1 = store(jj + 1, buf1, ssem1)
        s0.wait()
        gather(jj + 2, buf0, gsem0)
        s1.wait()
        gather(jj + 3, buf1, gsem1)

    pltpu.make_async_copy(in_hbm.at[idx_v.at[_NCH - 2]], buf0, gsem0).wait()
    s0 = store(_NCH - 2, buf0, ssem0)
    pltpu.make_async_copy(in_hbm.at[idx_v.at[_NCH - 1]], buf1, gsem1).wait()
    s1 = store(_NCH - 1, buf1, ssem1)
    s0.wait()
    s1.wait()


def kernel(chromatogram_batch):
    x = chromatogram_batch.transpose(1, 0, 2).reshape(_ROWS, _D)
    idx = jnp.asarray(_row_index_host())
    out = _shuffle(x, idx)
    return out.reshape(_C, _B, _D).transpose(1, 0, 2)
